# baseline (speedup 1.0000x reference)
import functools

import jax
import jax.numpy as jnp
from jax import lax
from jax.experimental import pallas as pl
from jax.experimental.pallas import tpu as pltpu, tpu_sc as plsc

VOCAB = 100000
EMB_DIM = 128
BATCH = 16384
N_FIELDS = 26
B_TOTAL = BATCH * N_FIELDS

_info = plsc.get_sparse_core_info()
_NC, _NS = _info.num_cores, _info.num_subcores
NW = _NC * _NS
FRAC_NUM, FRAC_DEN = 4, 5
B_SC = (B_TOTAL * FRAC_NUM // FRAC_DEN) // (NW * 8) * (NW * 8)
B_PER_W = B_SC // NW
NBUF = 4
CHUNK = 242

_mesh = plsc.VectorSubcoreMesh(core_axis_name="c", subcore_axis_name="s")


def _make(b_per_w, nbuf, chunk):
    nchunk = b_per_w // chunk
    npass = nchunk // nbuf

    @functools.partial(
        pl.kernel,
        mesh=_mesh,
        out_type=jax.ShapeDtypeStruct((b_per_w * NW, EMB_DIM), jnp.float32),
        scratch_types=(
            [pltpu.VMEM((b_per_w,), jnp.int32)]
            + [pltpu.VMEM((chunk, EMB_DIM), jnp.float32) for _ in range(nbuf)]
            + [pltpu.SemaphoreType.DMA for _ in range(2 * nbuf)]
        ),
    )
    def _sc_gather(idx_hbm, table_hbm, out_hbm, *refs):
        idx_v = refs[0]
        rows_v = refs[1 : 1 + nbuf]
        sg = refs[1 + nbuf : 1 + 2 * nbuf]
        so = refs[1 + 2 * nbuf : 1 + 3 * nbuf]

        wid = lax.axis_index("s") * _NC + lax.axis_index("c")
        base = wid * b_per_w

        pltpu.sync_copy(idx_hbm.at[pl.ds(base, b_per_w)], idx_v)

        def fire_gather(g, b):
            pltpu.async_copy(
                table_hbm.at[idx_v.at[pl.ds(g * chunk, chunk)]], rows_v[b], sg[b]
            )

        def wait_gather(b):
            pltpu.make_async_copy(
                table_hbm.at[idx_v.at[pl.ds(0, chunk)]], rows_v[b], sg[b]
            ).wait()

        def fire_out(g, b):
            pltpu.async_copy(
                rows_v[b], out_hbm.at[pl.ds(base + g * chunk, chunk)], so[b]
            )

        def wait_out(b):
            pltpu.make_async_copy(
                rows_v[b], out_hbm.at[pl.ds(base, chunk)], so[b]
            ).wait()

        fire_gather(0, 0)
        for b in range(1, nbuf):
            fire_gather(b, b)
            wait_gather(b - 1)
            fire_out(b - 1, b - 1)

        def body(p, carry):
            for b in range(nbuf):
                g = p * nbuf + b
                wait_out(b)
                fire_gather(g, b)
                b1 = (b - 1) % nbuf
                wait_gather(b1)
                fire_out(g - 1, b1)
            return carry

        lax.fori_loop(1, npass, body, 0)

        wait_gather(nbuf - 1)
        fire_out(nchunk - 1, nbuf - 1)
        for b in range(nbuf):
            wait_out(b)

    return _sc_gather


B_SC = 344064
_sc = _make(B_SC // NW, 4, 192)


def kernel(inputs, table):
    idx = inputs.reshape(-1).astype(jnp.int32)
    out_sc = _sc(idx[:B_SC], table)
    out_tc = jnp.take(table, idx[B_SC:], axis=0)
    out = jnp.concatenate([out_sc, out_tc], axis=0)
    return out.reshape(inputs.shape + (EMB_DIM,))


# speedup vs baseline: 1.2859x; 1.2859x over previous
import functools

import jax
import jax.numpy as jnp
from jax import lax
from jax.experimental import pallas as pl
from jax.experimental.pallas import tpu as pltpu, tpu_sc as plsc

VOCAB = 100000
EMB_DIM = 128
BATCH = 16384
N_FIELDS = 26
B_TOTAL = BATCH * N_FIELDS

_info = plsc.get_sparse_core_info()
_NC, _NS = _info.num_cores, _info.num_subcores
NW = _NC * _NS
B_PER_W = B_TOTAL // NW
NBUF = 4
CHUNK = 208
NCHUNK = B_PER_W // CHUNK
NP = NCHUNK // NBUF

_mesh = plsc.VectorSubcoreMesh(core_axis_name="c", subcore_axis_name="s")


@functools.partial(
    pl.kernel,
    mesh=_mesh,
    out_type=jax.ShapeDtypeStruct((B_TOTAL, EMB_DIM), jnp.float32),
    scratch_types=(
        [pltpu.VMEM((B_PER_W,), jnp.int32)]
        + [pltpu.VMEM((CHUNK, EMB_DIM), jnp.float32) for _ in range(NBUF)]
        + [pltpu.SemaphoreType.DMA for _ in range(2 * NBUF)]
    ),
)
def _sc_gather(idx_hbm, table_hbm, out_hbm, *refs):
    idx_v = refs[0]
    rows_v = refs[1 : 1 + NBUF]
    sg = refs[1 + NBUF : 1 + 2 * NBUF]
    so = refs[1 + 2 * NBUF : 1 + 3 * NBUF]

    wid = lax.axis_index("s") * _NC + lax.axis_index("c")
    base = wid * B_PER_W

    pltpu.sync_copy(idx_hbm.at[pl.ds(base, B_PER_W)], idx_v)

    def fire_gather(g, b):
        pltpu.async_copy(
            table_hbm.at[idx_v.at[pl.ds(g * CHUNK, CHUNK)]], rows_v[b], sg[b]
        )

    def wait_gather(b):
        pltpu.make_async_copy(
            table_hbm.at[idx_v.at[pl.ds(0, CHUNK)]], rows_v[b], sg[b]
        ).wait()

    def fire_out(g, b):
        pltpu.async_copy(rows_v[b], out_hbm.at[pl.ds(base + g * CHUNK, CHUNK)], so[b])

    def wait_out(b):
        pltpu.make_async_copy(rows_v[b], out_hbm.at[pl.ds(base, CHUNK)], so[b]).wait()

    fire_gather(0, 0)
    for b in range(1, NBUF):
        fire_gather(b, b)
        wait_gather(b - 1)
        fire_out(b - 1, b - 1)

    def body(p, carry):
        for b in range(NBUF):
            g = p * NBUF + b
            wait_out(b)
            fire_gather(g, b)
            b1 = (b - 1) % NBUF
            wait_gather(b1)
            fire_out(g - 1, b1)
        return carry

    lax.fori_loop(1, NP, body, 0)

    wait_gather(NBUF - 1)
    fire_out(NCHUNK - 1, NBUF - 1)
    for b in range(NBUF):
        wait_out(b)


def kernel(inputs, table):
    idx = inputs.reshape(-1).astype(jnp.int32)
    out = _sc_gather(idx, table)
    return out.reshape(inputs.shape + (EMB_DIM,))


# speedup vs baseline: 1.2889x; 1.0023x over previous
import functools

import jax
import jax.numpy as jnp
from jax import lax
from jax.experimental import pallas as pl
from jax.experimental.pallas import tpu as pltpu, tpu_sc as plsc

VOCAB = 100000
EMB_DIM = 128
BATCH = 16384
N_FIELDS = 26
B_TOTAL = BATCH * N_FIELDS

_info = plsc.get_sparse_core_info()
_NC, _NS = _info.num_cores, _info.num_subcores
NW = _NC * _NS
B_PER_W = B_TOTAL // NW
NBUF = 2
CHUNK = 416
NCHUNK = B_PER_W // CHUNK
NP = NCHUNK // NBUF

_mesh = plsc.VectorSubcoreMesh(core_axis_name="c", subcore_axis_name="s")


@functools.partial(
    pl.kernel,
    mesh=_mesh,
    out_type=jax.ShapeDtypeStruct((B_TOTAL, EMB_DIM), jnp.float32),
    scratch_types=(
        [pltpu.VMEM((B_PER_W,), jnp.int32)]
        + [pltpu.VMEM((CHUNK, EMB_DIM), jnp.float32) for _ in range(NBUF)]
        + [pltpu.SemaphoreType.DMA for _ in range(2 * NBUF)]
    ),
)
def _sc_gather(idx_hbm, table_hbm, out_hbm, *refs):
    idx_v = refs[0]
    rows_v = refs[1 : 1 + NBUF]
    sg = refs[1 + NBUF : 1 + 2 * NBUF]
    so = refs[1 + 2 * NBUF : 1 + 3 * NBUF]

    wid = lax.axis_index("s") * _NC + lax.axis_index("c")
    base = wid * B_PER_W

    pltpu.sync_copy(idx_hbm.at[pl.ds(base, B_PER_W)], idx_v)

    def fire_gather(g, b):
        pltpu.async_copy(
            table_hbm.at[idx_v.at[pl.ds(g * CHUNK, CHUNK)]], rows_v[b], sg[b]
        )

    def wait_gather(b):
        pltpu.make_async_copy(
            table_hbm.at[idx_v.at[pl.ds(0, CHUNK)]], rows_v[b], sg[b]
        ).wait()

    def fire_out(g, b):
        pltpu.async_copy(rows_v[b], out_hbm.at[pl.ds(base + g * CHUNK, CHUNK)], so[b])

    def wait_out(b):
        pltpu.make_async_copy(rows_v[b], out_hbm.at[pl.ds(base, CHUNK)], so[b]).wait()

    fire_gather(0, 0)
    for b in range(1, NBUF):
        fire_gather(b, b)
        wait_gather(b - 1)
        fire_out(b - 1, b - 1)

    def body(p, carry):
        for b in range(NBUF):
            g = p * NBUF + b
            wait_out(b)
            fire_gather(g, b)
            b1 = (b - 1) % NBUF
            wait_gather(b1)
            fire_out(g - 1, b1)
        return carry

    lax.fori_loop(1, NP, body, 0)

    wait_gather(NBUF - 1)
    fire_out(NCHUNK - 1, NBUF - 1)
    for b in range(NBUF):
        wait_out(b)


def kernel(inputs, table):
    idx = inputs.reshape(-1).astype(jnp.int32)
    out = _sc_gather(idx, table)
    return out.reshape(inputs.shape + (EMB_DIM,))
